# Initial kernel scaffold; baseline (speedup 1.0000x reference)
import jax
import jax.numpy as jnp
from jax.experimental import pallas as pl


def kernel(input_attenuation, input_wet_dry):
    raise NotImplementedError("write your pallas kernel here")


# speedup vs baseline: 5.3196x; 5.3196x over previous
import jax
import jax.numpy as jnp
from jax import lax
from jax.experimental import pallas as pl
from jax.experimental.pallas import tpu as pltpu
from jax.experimental.pallas import tpu_sc as plsc

N, S = 1024, 8192
L = 16
NC, NS = 2, 16
NW = NC * NS
ROWS_PER_W = N // NW
CHUNKS = S // L

_GDN = lax.GatherDimensionNumbers(
    offset_dims=(), collapsed_slice_dims=(0,), start_index_map=(0,))


def _gather16(v, idx):
    return lax.gather(v, idx[:, None], _GDN, slice_sizes=(1,),
                      mode=lax.GatherScatterMode.PROMISE_IN_BOUNDS)


def _bcast_last(v):
    return _gather16(v, jnp.full((L,), L - 1, jnp.int32))


def _ffill_body(attn_hbm, mask_hbm, out_hbm, attn_v, mask_v, out_v):
    wid = lax.axis_index("s") * NC + lax.axis_index("c")
    lane = lax.iota(jnp.int32, L)

    def do_row(r, _):
        pltpu.sync_copy(attn_hbm.at[r], attn_v)
        pltpu.sync_copy(mask_hbm.at[r], mask_v)
        carry0 = _gather16(attn_v[pl.ds(0, L)], jnp.zeros((L,), jnp.int32))

        def chunk(c, carry):
            a = attn_v[pl.ds(c * L, L)]
            m = mask_v[pl.ds(c * L, L)]
            didx = jnp.where(m == 0, lane, jnp.int32(-1))
            mx = plsc.cummax(didx)
            g = _gather16(a, jnp.maximum(mx, 0))
            res = jnp.where(mx >= 0, g, carry)
            out_v[pl.ds(c * L, L)] = res
            return _bcast_last(res)

        lax.fori_loop(0, CHUNKS, chunk, carry0)
        pltpu.sync_copy(out_v, out_hbm.at[r])
        return 0

    base = wid * ROWS_PER_W
    lax.fori_loop(base, base + ROWS_PER_W, do_row, 0)


def kernel(input_attenuation, input_wet_dry):
    mask = input_wet_dry.astype(jnp.int32)
    mesh = plsc.VectorSubcoreMesh(core_axis_name="c", subcore_axis_name="s")
    f = pl.kernel(
        _ffill_body,
        mesh=mesh,
        compiler_params=pltpu.CompilerParams(needs_layout_passes=False),
        out_type=jax.ShapeDtypeStruct((N, S), jnp.float32),
        scratch_types=[
            pltpu.VMEM((S,), jnp.float32),
            pltpu.VMEM((S,), jnp.int32),
            pltpu.VMEM((S,), jnp.float32),
        ],
    )
    return f(input_attenuation, mask)
